# baseline (speedup 1.0000x reference)
import functools

import jax
import jax.numpy as jnp
from jax import lax
from jax.experimental import pallas as pl
from jax.experimental.pallas import tpu as pltpu
from jax.experimental.pallas import tpu_sc as plsc

_B = 4096
_L = 50
_HID = 128
_N = _B * _L

_NC = 2
_NS = 16
_NW = _NC * _NS
_BPW = _B // _NW
_PER_W = _N // _NW
_CB = 2
_CHUNK = _CB * _L
_NCHUNK = _PER_W // _CHUNK
_NB = 4
_NGRP = _NCHUNK // _NB

_mesh = plsc.VectorSubcoreMesh(core_axis_name="c", subcore_axis_name="s")


@functools.partial(
    pl.kernel,
    mesh=_mesh,
    out_type=jax.ShapeDtypeStruct((_B, _L, _HID), jnp.float32),
    scratch_types=[
        pltpu.VMEM((_NCHUNK, _CHUNK), jnp.int32),
        pltpu.VMEM((_NB, _CHUNK, _HID), jnp.float32),
        pltpu.SemaphoreType.DMA((_NB,)),
        pltpu.SemaphoreType.DMA((_NB,)),
    ],
    compiler_params=pltpu.CompilerParams(use_tc_tiling_on_sc=True),
)
def _gather_kernel(src_hbm, table_hbm, out_hbm, idx_v, rows_v, gsem, wsem):
    wid = lax.axis_index("s") * _NC + lax.axis_index("c")
    base_b = wid * _BPW
    pltpu.sync_copy(src_hbm.at[wid], idx_v)

    def fire_writebacks(c, b):
        bb = base_b + c * _CB
        for j in range(_CB):
            pltpu.async_copy(
                rows_v.at[b, pl.ds(j * _L, _L)],
                out_hbm.at[bb + j],
                wsem.at[b],
            )

    def wait_writebacks(c, b):
        bb = base_b + c * _CB
        for j in range(_CB):
            pltpu.make_async_copy(
                rows_v.at[b, pl.ds(j * _L, _L)],
                out_hbm.at[bb + j],
                wsem.at[b],
            ).wait()

    def wait_gather(c, b):
        pltpu.make_async_copy(
            table_hbm.at[idx_v.at[c]], rows_v.at[b], gsem.at[b]
        ).wait()

    for b in range(_NB):
        pltpu.async_copy(table_hbm.at[idx_v.at[b]], rows_v.at[b], gsem.at[b])

    def group(o, carry):
        for b in range(_NB):
            c = o * _NB + b
            wait_gather(c, b)
            fire_writebacks(c, b)
        for b in range(_NB):
            c = o * _NB + b
            wait_writebacks(c, b)
            pltpu.async_copy(
                table_hbm.at[idx_v.at[(o + 1) * _NB + b]],
                rows_v.at[b],
                gsem.at[b],
            )
        return carry

    lax.fori_loop(0, _NGRP - 1, group, 0)

    for b in range(_NB):
        c = (_NGRP - 1) * _NB + b
        wait_gather(c, b)
        fire_writebacks(c, b)
    for b in range(_NB):
        c = (_NGRP - 1) * _NB + b
        wait_writebacks(c, b)


def kernel(source, table):
    src = source.reshape(_NW, _NCHUNK, _CHUNK).astype(jnp.int32)
    return _gather_kernel(src, table)


# speedup vs baseline: 1.0006x; 1.0006x over previous
import functools

import jax
import jax.numpy as jnp
from jax import lax
from jax.experimental import pallas as pl
from jax.experimental.pallas import tpu as pltpu
from jax.experimental.pallas import tpu_sc as plsc

_B = 4096
_L = 50
_HID = 128
_N = _B * _L

_NC = 2
_NS = 16
_NW = _NC * _NS
_BPW = _B // _NW
_PER_W = _N // _NW
_CB = 2
_CHUNK = _CB * _L
_NCHUNK = _PER_W // _CHUNK
_NB = 4
_NGRP = _NCHUNK // _NB

_mesh = plsc.VectorSubcoreMesh(core_axis_name="c", subcore_axis_name="s")


@functools.partial(
    pl.kernel,
    mesh=_mesh,
    out_type=jax.ShapeDtypeStruct((_B, _L, _HID), jnp.float32),
    scratch_types=[
        pltpu.VMEM((_NCHUNK, _CHUNK), jnp.int32),
        pltpu.VMEM((_NB, _CHUNK, _HID), jnp.float32),
        pltpu.SemaphoreType.DMA((_NB,)),
        pltpu.SemaphoreType.DMA((_NB,)),
    ],
    compiler_params=pltpu.CompilerParams(
        use_tc_tiling_on_sc=True, needs_layout_passes=True
    ),
)
def _gather_kernel(src_hbm, table_hbm, out_hbm, idx_v, rows_v, gsem, wsem):
    wid = lax.axis_index("s") * _NC + lax.axis_index("c")
    base_b = wid * _BPW
    pltpu.sync_copy(src_hbm.at[wid], idx_v)

    def fire_writebacks(c, b):
        bb = base_b + c * _CB
        for j in range(_CB):
            pltpu.async_copy(
                rows_v.at[b, pl.ds(j * _L, _L)],
                out_hbm.at[bb + j],
                wsem.at[b],
            )

    def wait_writebacks(c, b):
        bb = base_b + c * _CB
        for j in range(_CB):
            pltpu.make_async_copy(
                rows_v.at[b, pl.ds(j * _L, _L)],
                out_hbm.at[bb + j],
                wsem.at[b],
            ).wait()

    def wait_gather(c, b):
        pltpu.make_async_copy(
            table_hbm.at[idx_v.at[c]], rows_v.at[b], gsem.at[b]
        ).wait()

    for b in range(_NB):
        pltpu.async_copy(table_hbm.at[idx_v.at[b]], rows_v.at[b], gsem.at[b])

    def group(o, carry):
        for b in range(_NB):
            c = o * _NB + b
            wait_gather(c, b)
            fire_writebacks(c, b)
        for b in range(_NB):
            c = o * _NB + b
            wait_writebacks(c, b)
            pltpu.async_copy(
                table_hbm.at[idx_v.at[(o + 1) * _NB + b]],
                rows_v.at[b],
                gsem.at[b],
            )
        return carry

    lax.fori_loop(0, _NGRP - 1, group, 0)

    for b in range(_NB):
        c = (_NGRP - 1) * _NB + b
        wait_gather(c, b)
        fire_writebacks(c, b)
    for b in range(_NB):
        c = (_NGRP - 1) * _NB + b
        wait_writebacks(c, b)


def kernel(source, table):
    src = source.reshape(_NW, _NCHUNK, _CHUNK).astype(jnp.int32)
    return _gather_kernel(src, table)
